# baseline (speedup 1.0000x reference)
import jax
import jax.numpy as jnp
from jax.experimental import pallas as pl

N_FPS_ = 2048
N_KNN_ = 20


def _fps(xyz, n_fps):
    B, N, _ = xyz.shape

    def body(i, state):
        idxs, dists, farthest = state
        idxs = idxs.at[:, i].set(farthest)
        centroid = jnp.take_along_axis(xyz, farthest[:, None, None], axis=1)
        d = jnp.sum((xyz - centroid) ** 2, axis=-1)
        dists = jnp.minimum(dists, d)
        farthest = jnp.argmax(dists, axis=-1).astype(jnp.int32)
        return (idxs, dists, farthest)

    idxs0 = jnp.zeros((B, n_fps), dtype=jnp.int32)
    dists0 = jnp.full((B, N), 1e10, dtype=jnp.float32)
    farthest0 = jnp.zeros((B,), dtype=jnp.int32)
    idxs, _, _ = jax.lax.fori_loop(0, n_fps, body, (idxs0, dists0, farthest0))
    return idxs


def _knn_idx(query, points, k):
    q2 = jnp.sum(query ** 2, axis=-1)[:, :, None]
    p2 = jnp.sum(points ** 2, axis=-1)[:, None, :]
    d = q2 + p2 - 2.0 * jnp.einsum('bsd,bnd->bsn', query, points)
    _, idx = jax.lax.top_k(-d, k)
    return idx


def _sub_kernel(g_ref, c_ref, o_ref):
    o_ref[...] = g_ref[...] - c_ref[...]


def kernel(xyz, features):
    B, N, _ = xyz.shape
    C = features.shape[1]
    S, K = N_FPS_, N_KNN_

    center_idx = _fps(jax.lax.stop_gradient(xyz), S)
    bidx = jnp.arange(B)[:, None]
    FPS_xyz = xyz[bidx, center_idx]
    feats_t = jnp.transpose(features, (0, 2, 1))
    center_features = jnp.transpose(feats_t[bidx, center_idx], (0, 2, 1))
    idx = _knn_idx(FPS_xyz, xyz, K)
    bidx3 = jnp.arange(B)[:, None, None]
    grouped_xyz = jnp.transpose(xyz[bidx3, idx], (0, 3, 1, 2))
    grouped_features = jnp.transpose(feats_t[bidx3, idx], (0, 3, 1, 2))

    g = jnp.concatenate([grouped_xyz, grouped_features], axis=1)
    c = jnp.concatenate(
        [jnp.transpose(FPS_xyz, (0, 2, 1)), center_features], axis=1)
    c_rep = jnp.repeat(c[:, :, :, None], K, axis=3)

    CH = 3 + C
    SB = 128
    g2 = g.reshape(B, CH, S * K)
    c2 = c_rep.reshape(B, CH, S * K)
    graph_features = pl.pallas_call(
        _sub_kernel,
        grid=(B, S // SB),
        in_specs=[
            pl.BlockSpec((1, CH, SB * K), lambda b, s: (b, 0, s)),
            pl.BlockSpec((1, CH, SB * K), lambda b, s: (b, 0, s)),
        ],
        out_specs=pl.BlockSpec((1, CH, SB * K), lambda b, s: (b, 0, s)),
        out_shape=jax.ShapeDtypeStruct((B, CH, S * K), jnp.float32),
    )(g2, c2)
    return (FPS_xyz, graph_features.reshape(B, CH, S, K))


# speedup vs baseline: 1.6929x; 1.6929x over previous
import functools

import jax
import jax.numpy as jnp
from jax.experimental import pallas as pl
from jax.experimental.pallas import tpu as pltpu

N_FPS_ = 2048
N_KNN_ = 20
_NR, _NL = 64, 128


def _fps_body(x_ref, idx_ref, fxyz_ref, dists_ref):
    x = x_ref[0, 0]
    y = x_ref[0, 1]
    z = x_ref[0, 2]
    rows = jax.lax.broadcasted_iota(jnp.int32, (_NR, _NL), 0)
    cols = jax.lax.broadcasted_iota(jnp.int32, (_NR, _NL), 1)
    lin = rows * _NL + cols
    dists_ref[...] = jnp.full((_NR, _NL), 1e10, jnp.float32)

    def body(i, f):
        mask = lin == f
        neg = jnp.float32(-3e38)
        cx = jnp.max(jnp.where(mask, x, neg))
        cy = jnp.max(jnp.where(mask, y, neg))
        cz = jnp.max(jnp.where(mask, z, neg))
        idx_ref[0, 0, i] = f
        fxyz_ref[0, 0, i] = cx
        fxyz_ref[0, 1, i] = cy
        fxyz_ref[0, 2, i] = cz
        dx = x - cx
        dy = y - cy
        dz = z - cz
        d = (dx * dx + dy * dy) + dz * dz
        nd = jnp.minimum(dists_ref[...], d)
        dists_ref[...] = nd
        m = jnp.max(nd)
        fnext = jnp.min(jnp.where(nd == m, lin, jnp.int32(2 ** 30)))
        return fnext

    jax.lax.fori_loop(0, N_FPS_, body, jnp.int32(0))


def _fps_pallas(xyz):
    B, N, _ = xyz.shape
    xt = xyz.transpose(0, 2, 1).reshape(B, 3, _NR, _NL)
    idx, fxyz = pl.pallas_call(
        _fps_body,
        grid=(B,),
        in_specs=[pl.BlockSpec((1, 3, _NR, _NL), lambda b: (b, 0, 0, 0))],
        out_specs=[
            pl.BlockSpec((1, 1, N_FPS_), lambda b: (b, 0, 0),
                         memory_space=pltpu.SMEM),
            pl.BlockSpec((1, 3, N_FPS_), lambda b: (b, 0, 0),
                         memory_space=pltpu.SMEM),
        ],
        out_shape=[
            jax.ShapeDtypeStruct((B, 1, N_FPS_), jnp.int32),
            jax.ShapeDtypeStruct((B, 3, N_FPS_), jnp.float32),
        ],
        scratch_shapes=[pltpu.VMEM((_NR, _NL), jnp.float32)],
    )(xt)
    return idx.reshape(B, N_FPS_), fxyz.transpose(0, 2, 1)


def _knn_idx(query, points, k):
    q2 = jnp.sum(query ** 2, axis=-1)[:, :, None]
    p2 = jnp.sum(points ** 2, axis=-1)[:, None, :]
    d = q2 + p2 - 2.0 * jnp.einsum('bsd,bnd->bsn', query, points)
    _, idx = jax.lax.top_k(-d, k)
    return idx


def _sub_kernel(g_ref, c_ref, o_ref):
    o_ref[...] = g_ref[...] - c_ref[...]


def kernel(xyz, features):
    B, N, _ = xyz.shape
    C = features.shape[1]
    S, K = N_FPS_, N_KNN_

    center_idx, FPS_xyz = _fps_pallas(jax.lax.stop_gradient(xyz))
    bidx = jnp.arange(B)[:, None]
    feats_t = jnp.transpose(features, (0, 2, 1))
    center_features = jnp.transpose(feats_t[bidx, center_idx], (0, 2, 1))
    idx = _knn_idx(FPS_xyz, xyz, K)
    bidx3 = jnp.arange(B)[:, None, None]
    grouped_xyz = jnp.transpose(xyz[bidx3, idx], (0, 3, 1, 2))
    grouped_features = jnp.transpose(feats_t[bidx3, idx], (0, 3, 1, 2))

    g = jnp.concatenate([grouped_xyz, grouped_features], axis=1)
    c = jnp.concatenate(
        [jnp.transpose(FPS_xyz, (0, 2, 1)), center_features], axis=1)
    c_rep = jnp.repeat(c[:, :, :, None], K, axis=3)

    CH = 3 + C
    SB = 128
    g2 = g.reshape(B, CH, S * K)
    c2 = c_rep.reshape(B, CH, S * K)
    graph_features = pl.pallas_call(
        _sub_kernel,
        grid=(B, S // SB),
        in_specs=[
            pl.BlockSpec((1, CH, SB * K), lambda b, s: (b, 0, s)),
            pl.BlockSpec((1, CH, SB * K), lambda b, s: (b, 0, s)),
        ],
        out_specs=pl.BlockSpec((1, CH, SB * K), lambda b, s: (b, 0, s)),
        out_shape=jax.ShapeDtypeStruct((B, CH, S * K), jnp.float32),
    )(g2, c2)
    return (FPS_xyz, graph_features.reshape(B, CH, S, K))


# speedup vs baseline: 4.0196x; 2.3744x over previous
import functools

import jax
import jax.numpy as jnp
from jax.experimental import pallas as pl
from jax.experimental.pallas import tpu as pltpu

N_FPS_ = 2048
N_KNN_ = 20
_NR, _NL = 64, 128


def _fps_body(x_ref, idx_ref, fxyz_ref, dists_ref):
    x = x_ref[0, 0]
    y = x_ref[0, 1]
    z = x_ref[0, 2]
    rows = jax.lax.broadcasted_iota(jnp.int32, (_NR, _NL), 0)
    cols = jax.lax.broadcasted_iota(jnp.int32, (_NR, _NL), 1)
    lin = rows * _NL + cols
    dists_ref[...] = jnp.full((_NR, _NL), 1e10, jnp.float32)

    def body(i, f):
        mask = lin == f
        neg = jnp.float32(-3e38)
        cx = jnp.max(jnp.where(mask, x, neg))
        cy = jnp.max(jnp.where(mask, y, neg))
        cz = jnp.max(jnp.where(mask, z, neg))
        idx_ref[0, 0, i] = f
        fxyz_ref[0, 0, i] = cx
        fxyz_ref[0, 1, i] = cy
        fxyz_ref[0, 2, i] = cz
        dx = x - cx
        dy = y - cy
        dz = z - cz
        d = (dx * dx + dy * dy) + dz * dz
        nd = jnp.minimum(dists_ref[...], d)
        dists_ref[...] = nd
        m = jnp.max(nd)
        fnext = jnp.min(jnp.where(nd == m, lin, jnp.int32(2 ** 30)))
        return fnext

    jax.lax.fori_loop(0, N_FPS_, body, jnp.int32(0))


def _fps_pallas(xyz):
    B, N, _ = xyz.shape
    xt = xyz.transpose(0, 2, 1).reshape(B, 3, _NR, _NL)
    idx, fxyz = pl.pallas_call(
        _fps_body,
        grid=(B,),
        in_specs=[pl.BlockSpec((1, 3, _NR, _NL), lambda b: (b, 0, 0, 0))],
        out_specs=[
            pl.BlockSpec((1, 1, N_FPS_), lambda b: (b, 0, 0),
                         memory_space=pltpu.SMEM),
            pl.BlockSpec((1, 3, N_FPS_), lambda b: (b, 0, 0),
                         memory_space=pltpu.SMEM),
        ],
        out_shape=[
            jax.ShapeDtypeStruct((B, 1, N_FPS_), jnp.int32),
            jax.ShapeDtypeStruct((B, 3, N_FPS_), jnp.float32),
        ],
        scratch_shapes=[pltpu.VMEM((_NR, _NL), jnp.float32)],
    )(xt)
    return idx.reshape(B, N_FPS_), fxyz


_QT = 128


def _knn_body(p_ref, q_ref, idx_ref, d_ref):
    p = p_ref[0]
    q = q_ref[0]
    px, py, pz = p[0:1, :], p[1:2, :], p[2:3, :]
    qx, qy, qz = q[:, 0:1], q[:, 1:2], q[:, 2:3]
    p2 = (px * px + py * py) + pz * pz
    q2 = (qx * qx + qy * qy) + qz * qz
    bf, f32 = jnp.bfloat16, jnp.float32
    pxb, pyb, pzb = (px.astype(bf).astype(f32), py.astype(bf).astype(f32),
                     pz.astype(bf).astype(f32))
    qxb, qyb, qzb = (qx.astype(bf).astype(f32), qy.astype(bf).astype(f32),
                     qz.astype(bf).astype(f32))
    qp = (qxb * pxb + qyb * pyb) + qzb * pzb
    d_ref[...] = (q2 + p2) - 2.0 * qp
    lin = jax.lax.broadcasted_iota(jnp.int32, (_QT, 8192), 1)
    big = jnp.int32(2 ** 30)
    inf = jnp.float32(3e38)
    for kk in range(N_KNN_):
        d = d_ref[...]
        m = jnp.min(d, axis=1, keepdims=True)
        ai = jnp.min(jnp.where(d == m, lin, big), axis=1, keepdims=True)
        idx_ref[0, :, kk:kk + 1] = ai
        d_ref[...] = jnp.where(lin == ai, inf, d)


def _knn_pallas(fxyz_t, xyz):
    B, N, _ = xyz.shape
    S = N_FPS_
    pt = jnp.concatenate(
        [xyz.transpose(0, 2, 1),
         jnp.zeros((B, 5, N), jnp.float32)], axis=1)
    qt = jnp.concatenate(
        [fxyz_t.transpose(0, 2, 1),
         jnp.zeros((B, S, 5), jnp.float32)], axis=2)
    idx = pl.pallas_call(
        _knn_body,
        grid=(B, S // _QT),
        in_specs=[
            pl.BlockSpec((1, 8, N), lambda b, s: (b, 0, 0)),
            pl.BlockSpec((1, _QT, 8), lambda b, s: (b, s, 0)),
        ],
        out_specs=pl.BlockSpec((1, _QT, N_KNN_), lambda b, s: (b, s, 0)),
        out_shape=jax.ShapeDtypeStruct((B, S, N_KNN_), jnp.int32),
        scratch_shapes=[pltpu.VMEM((_QT, N), jnp.float32)],
    )(pt, qt)
    return idx


def _sub_kernel(g_ref, c_ref, o_ref):
    o_ref[...] = g_ref[...] - c_ref[...]


def kernel(xyz, features):
    B, N, _ = xyz.shape
    C = features.shape[1]
    S, K = N_FPS_, N_KNN_

    center_idx, fxyz_t = _fps_pallas(jax.lax.stop_gradient(xyz))
    FPS_xyz = fxyz_t.transpose(0, 2, 1)
    bidx = jnp.arange(B)[:, None]
    feats_t = jnp.transpose(features, (0, 2, 1))
    center_features = jnp.transpose(feats_t[bidx, center_idx], (0, 2, 1))
    idx = _knn_pallas(fxyz_t, xyz)
    bidx3 = jnp.arange(B)[:, None, None]
    grouped_xyz = jnp.transpose(xyz[bidx3, idx], (0, 3, 1, 2))
    grouped_features = jnp.transpose(feats_t[bidx3, idx], (0, 3, 1, 2))

    g = jnp.concatenate([grouped_xyz, grouped_features], axis=1)
    c = jnp.concatenate(
        [jnp.transpose(FPS_xyz, (0, 2, 1)), center_features], axis=1)
    c_rep = jnp.repeat(c[:, :, :, None], K, axis=3)

    CH = 3 + C
    SB = 128
    g2 = g.reshape(B, CH, S * K)
    c2 = c_rep.reshape(B, CH, S * K)
    graph_features = pl.pallas_call(
        _sub_kernel,
        grid=(B, S // SB),
        in_specs=[
            pl.BlockSpec((1, CH, SB * K), lambda b, s: (b, 0, s)),
            pl.BlockSpec((1, CH, SB * K), lambda b, s: (b, 0, s)),
        ],
        out_specs=pl.BlockSpec((1, CH, SB * K), lambda b, s: (b, 0, s)),
        out_shape=jax.ShapeDtypeStruct((B, CH, S * K), jnp.float32),
    )(g2, c2)
    return (FPS_xyz, graph_features.reshape(B, CH, S, K))


# speedup vs baseline: 7.9122x; 1.9684x over previous
import functools

import jax
import jax.numpy as jnp
from jax.experimental import pallas as pl
from jax.experimental.pallas import tpu as pltpu

N_FPS_ = 2048
N_KNN_ = 20
_NR, _NL = 64, 128


def _fps_body(x_ref, idx_ref, fxyz_ref, dists_ref):
    x = x_ref[0, 0]
    y = x_ref[0, 1]
    z = x_ref[0, 2]
    rows = jax.lax.broadcasted_iota(jnp.int32, (_NR, _NL), 0)
    cols = jax.lax.broadcasted_iota(jnp.int32, (_NR, _NL), 1)
    lin = rows * _NL + cols
    dists_ref[...] = jnp.full((_NR, _NL), 1e10, jnp.float32)

    def body(i, f):
        mask = lin == f
        neg = jnp.float32(-3e38)
        cx = jnp.max(jnp.where(mask, x, neg))
        cy = jnp.max(jnp.where(mask, y, neg))
        cz = jnp.max(jnp.where(mask, z, neg))
        idx_ref[0, 0, i] = f
        fxyz_ref[0, 0, i] = cx
        fxyz_ref[0, 1, i] = cy
        fxyz_ref[0, 2, i] = cz
        dx = x - cx
        dy = y - cy
        dz = z - cz
        d = (dx * dx + dy * dy) + dz * dz
        nd = jnp.minimum(dists_ref[...], d)
        dists_ref[...] = nd
        m = jnp.max(nd)
        fnext = jnp.min(jnp.where(nd == m, lin, jnp.int32(2 ** 30)))
        return fnext

    jax.lax.fori_loop(0, N_FPS_, body, jnp.int32(0))


def _fps_pallas(xyz):
    B, N, _ = xyz.shape
    xt = xyz.transpose(0, 2, 1).reshape(B, 3, _NR, _NL)
    idx, fxyz = pl.pallas_call(
        _fps_body,
        grid=(B,),
        in_specs=[pl.BlockSpec((1, 3, _NR, _NL), lambda b: (b, 0, 0, 0))],
        out_specs=[
            pl.BlockSpec((1, 1, N_FPS_), lambda b: (b, 0, 0),
                         memory_space=pltpu.SMEM),
            pl.BlockSpec((1, 3, N_FPS_), lambda b: (b, 0, 0),
                         memory_space=pltpu.SMEM),
        ],
        out_shape=[
            jax.ShapeDtypeStruct((B, 1, N_FPS_), jnp.int32),
            jax.ShapeDtypeStruct((B, 3, N_FPS_), jnp.float32),
        ],
        scratch_shapes=[pltpu.VMEM((_NR, _NL), jnp.float32)],
    )(xt)
    return idx.reshape(B, N_FPS_), fxyz


_QT = 128


def _knn_body(p_ref, q_ref, idx_ref, d_ref):
    p = p_ref[0]
    q = q_ref[0]
    px, py, pz = p[0:1, :], p[1:2, :], p[2:3, :]
    qx, qy, qz = q[:, 0:1], q[:, 1:2], q[:, 2:3]
    p2 = (px * px + py * py) + pz * pz
    q2 = (qx * qx + qy * qy) + qz * qz
    bf, f32 = jnp.bfloat16, jnp.float32
    pxb, pyb, pzb = (px.astype(bf).astype(f32), py.astype(bf).astype(f32),
                     pz.astype(bf).astype(f32))
    qxb, qyb, qzb = (qx.astype(bf).astype(f32), qy.astype(bf).astype(f32),
                     qz.astype(bf).astype(f32))
    qp = (qxb * pxb + qyb * pyb) + qzb * pzb
    d_ref[...] = (q2 + p2) - 2.0 * qp
    lin = jax.lax.broadcasted_iota(jnp.int32, (_QT, 8192), 1)
    big = jnp.int32(2 ** 30)
    inf = jnp.float32(3e38)
    for kk in range(N_KNN_):
        d = d_ref[...]
        m = jnp.min(d, axis=1, keepdims=True)
        ai = jnp.min(jnp.where(d == m, lin, big), axis=1, keepdims=True)
        idx_ref[0, :, kk:kk + 1] = ai
        d_ref[...] = jnp.where(lin == ai, inf, d)


def _knn_pallas(fxyz_t, xyz):
    B, N, _ = xyz.shape
    S = N_FPS_
    pt = jnp.concatenate(
        [xyz.transpose(0, 2, 1),
         jnp.zeros((B, 5, N), jnp.float32)], axis=1)
    qt = jnp.concatenate(
        [fxyz_t.transpose(0, 2, 1),
         jnp.zeros((B, S, 5), jnp.float32)], axis=2)
    idx = pl.pallas_call(
        _knn_body,
        grid=(B, S // _QT),
        in_specs=[
            pl.BlockSpec((1, 8, N), lambda b, s: (b, 0, 0)),
            pl.BlockSpec((1, _QT, 8), lambda b, s: (b, s, 0)),
        ],
        out_specs=pl.BlockSpec((1, _QT, N_KNN_), lambda b, s: (b, s, 0)),
        out_shape=jax.ShapeDtypeStruct((B, S, N_KNN_), jnp.int32),
        scratch_shapes=[pltpu.VMEM((_QT, N), jnp.float32)],
    )(pt, qt)
    return idx


_CH = 131
_TPB = 8
_SKW = N_FPS_ * N_KNN_ // _TPB


def _sc_gather_sub(B, N):
    from jax import lax
    from jax.experimental.pallas import tpu_sc as plsc

    mesh = plsc.VectorSubcoreMesh(core_axis_name="c", subcore_axis_name="s")

    @functools.partial(
        pl.kernel, mesh=mesh,
        compiler_params=pltpu.CompilerParams(needs_layout_passes=False),
        out_type=jax.ShapeDtypeStruct((B, _CH, _TPB * _SKW), jnp.float32),
        scratch_types=[
            pltpu.VMEM((N,), jnp.float32),
            pltpu.VMEM((_SKW,), jnp.int32),
            pltpu.VMEM((_SKW,), jnp.int32),
            pltpu.VMEM((_SKW,), jnp.float32),
        ],
    )
    def k(pf_hbm, idxf_hbm, cidx_hbm, out_hbm, colv, idxv, cidxv, outv):
        wid = lax.axis_index("s") * 2 + lax.axis_index("c")
        b = wid // _TPB
        base = (wid % _TPB) * _SKW
        pltpu.sync_copy(idxf_hbm.at[b, pl.ds(base, _SKW)], idxv)
        pltpu.sync_copy(cidx_hbm.at[b, pl.ds(base, _SKW)], cidxv)

        def chan(c, _):
            pltpu.sync_copy(pf_hbm.at[b, c], colv)

            def vec(j, _):
                iv = idxv[pl.ds(j * 16, 16)]
                cv = cidxv[pl.ds(j * 16, 16)]
                a = plsc.load_gather(colv, [iv])
                ctr = plsc.load_gather(colv, [cv])
                outv[pl.ds(j * 16, 16)] = a - ctr
                return 0

            lax.fori_loop(0, _SKW // 16, vec, 0, unroll=8)
            pltpu.sync_copy(outv, out_hbm.at[b, c, pl.ds(base, _SKW)])
            return 0

        lax.fori_loop(0, _CH, chan, 0)

    return k


def kernel(xyz, features):
    B, N, _ = xyz.shape
    C = features.shape[1]
    S, K = N_FPS_, N_KNN_

    center_idx, fxyz_t = _fps_pallas(jax.lax.stop_gradient(xyz))
    FPS_xyz = fxyz_t.transpose(0, 2, 1)
    idx = _knn_pallas(fxyz_t, xyz)

    pf = jnp.concatenate([xyz.transpose(0, 2, 1), features], axis=1)
    idxf = idx.reshape(B, S * K)
    cidxf = jnp.repeat(center_idx, K, axis=1)
    out = _sc_gather_sub(B, N)(pf, idxf, cidxf)
    return (FPS_xyz, out.reshape(B, _CH, S, K))


# speedup vs baseline: 9.0219x; 1.1403x over previous
import functools

import jax
import jax.numpy as jnp
from jax.experimental import pallas as pl
from jax.experimental.pallas import tpu as pltpu

N_FPS_ = 2048
N_KNN_ = 20
_NR, _NL = 64, 128


def _fps_body_all(x_ref, idx_ref, fxyz_ref, dists_ref):
    B = idx_ref.shape[0]
    rows = jax.lax.broadcasted_iota(jnp.int32, (_NR, _NL), 0)
    cols = jax.lax.broadcasted_iota(jnp.int32, (_NR, _NL), 1)
    lin = rows * _NL + cols
    neg = jnp.float32(-3e38)
    big = jnp.int32(2 ** 30)
    for b in range(B):
        dists_ref[b] = jnp.full((_NR, _NL), 1e10, jnp.float32)

    def body(i, fs):
        nfs = []
        for b in range(B):
            f = fs[b]
            x = x_ref[0, b]
            y = x_ref[1, b]
            z = x_ref[2, b]
            mask = lin == f
            cx = jnp.max(jnp.where(mask, x, neg))
            cy = jnp.max(jnp.where(mask, y, neg))
            cz = jnp.max(jnp.where(mask, z, neg))
            idx_ref[b, 0, i] = f
            fxyz_ref[b, 0, i] = cx
            fxyz_ref[b, 1, i] = cy
            fxyz_ref[b, 2, i] = cz
            dx = x - cx
            dy = y - cy
            dz = z - cz
            d = (dx * dx + dy * dy) + dz * dz
            nd = jnp.minimum(dists_ref[b], d)
            dists_ref[b] = nd
            m = jnp.max(nd)
            nfs.append(jnp.min(jnp.where(nd == m, lin, big)))
        return tuple(nfs)

    jax.lax.fori_loop(0, N_FPS_, body, (jnp.int32(0),) * B)


def _fps_pallas(xyz):
    B, N, _ = xyz.shape
    xt = xyz.reshape(B, _NR, _NL, 3).transpose(3, 0, 1, 2)
    idx, fxyz = pl.pallas_call(
        _fps_body_all,
        grid=(1,),
        in_specs=[pl.BlockSpec((3, B, _NR, _NL), lambda i: (0, 0, 0, 0))],
        out_specs=[
            pl.BlockSpec((B, 1, N_FPS_), lambda i: (0, 0, 0),
                         memory_space=pltpu.SMEM),
            pl.BlockSpec((B, 3, N_FPS_), lambda i: (0, 0, 0),
                         memory_space=pltpu.SMEM),
        ],
        out_shape=[
            jax.ShapeDtypeStruct((B, 1, N_FPS_), jnp.int32),
            jax.ShapeDtypeStruct((B, 3, N_FPS_), jnp.float32),
        ],
        scratch_shapes=[pltpu.VMEM((B, _NR, _NL), jnp.float32)],
    )(xt)
    return idx.reshape(B, N_FPS_), fxyz


_QT = 128


def _knn_body(p_ref, q_ref, idx_ref, d_ref):
    p = p_ref[0]
    q = q_ref[0]
    px, py, pz = p[0:1, :], p[1:2, :], p[2:3, :]
    qx, qy, qz = q[:, 0:1], q[:, 1:2], q[:, 2:3]
    p2 = (px * px + py * py) + pz * pz
    q2 = (qx * qx + qy * qy) + qz * qz
    bf, f32 = jnp.bfloat16, jnp.float32
    pxb, pyb, pzb = (px.astype(bf).astype(f32), py.astype(bf).astype(f32),
                     pz.astype(bf).astype(f32))
    qxb, qyb, qzb = (qx.astype(bf).astype(f32), qy.astype(bf).astype(f32),
                     qz.astype(bf).astype(f32))
    qp = (qxb * pxb + qyb * pyb) + qzb * pzb
    d_ref[...] = (q2 + p2) - 2.0 * qp
    lin = jax.lax.broadcasted_iota(jnp.int32, (_QT, 8192), 1)
    big = jnp.int32(2 ** 30)
    inf = jnp.float32(3e38)
    for kk in range(N_KNN_):
        d = d_ref[...]
        m = jnp.min(d, axis=1, keepdims=True)
        ai = jnp.min(jnp.where(d == m, lin, big), axis=1, keepdims=True)
        idx_ref[0, :, kk:kk + 1] = ai
        d_ref[...] = jnp.where(lin == ai, inf, d)


def _knn_pallas(fxyz_t, xyz):
    B, N, _ = xyz.shape
    S = N_FPS_
    pt = jnp.concatenate(
        [xyz.transpose(0, 2, 1),
         jnp.zeros((B, 5, N), jnp.float32)], axis=1)
    qt = jnp.concatenate(
        [fxyz_t.transpose(0, 2, 1),
         jnp.zeros((B, S, 5), jnp.float32)], axis=2)
    idx = pl.pallas_call(
        _knn_body,
        grid=(B, S // _QT),
        in_specs=[
            pl.BlockSpec((1, 8, N), lambda b, s: (b, 0, 0)),
            pl.BlockSpec((1, _QT, 8), lambda b, s: (b, s, 0)),
        ],
        out_specs=pl.BlockSpec((1, _QT, N_KNN_), lambda b, s: (b, s, 0)),
        out_shape=jax.ShapeDtypeStruct((B, S, N_KNN_), jnp.int32),
        scratch_shapes=[pltpu.VMEM((_QT, N), jnp.float32)],
    )(pt, qt)
    return idx


_CH = 131
_TPB = 8
_SKW = N_FPS_ * N_KNN_ // _TPB


def _sc_gather_sub(B, N):
    from jax import lax
    from jax.experimental.pallas import tpu_sc as plsc

    mesh = plsc.VectorSubcoreMesh(core_axis_name="c", subcore_axis_name="s")

    @functools.partial(
        pl.kernel, mesh=mesh,
        compiler_params=pltpu.CompilerParams(needs_layout_passes=False),
        out_type=jax.ShapeDtypeStruct((B, _CH, _TPB * _SKW), jnp.float32),
        scratch_types=[
            pltpu.VMEM((N,), jnp.float32),
            pltpu.VMEM((_SKW,), jnp.int32),
            pltpu.VMEM((_SKW,), jnp.int32),
            pltpu.VMEM((_SKW,), jnp.float32),
        ],
    )
    def k(pf_hbm, idxf_hbm, cidx_hbm, out_hbm, colv, idxv, cidxv, outv):
        wid = lax.axis_index("s") * 2 + lax.axis_index("c")
        b = wid // _TPB
        base = (wid % _TPB) * _SKW
        pltpu.sync_copy(idxf_hbm.at[b, pl.ds(base, _SKW)], idxv)
        pltpu.sync_copy(cidx_hbm.at[b, pl.ds(base, _SKW)], cidxv)

        def chan(c, _):
            pltpu.sync_copy(pf_hbm.at[b, c], colv)

            def vec(j, _):
                iv = idxv[pl.ds(j * 16, 16)]
                cv = cidxv[pl.ds(j * 16, 16)]
                a = plsc.load_gather(colv, [iv])
                ctr = plsc.load_gather(colv, [cv])
                outv[pl.ds(j * 16, 16)] = a - ctr
                return 0

            lax.fori_loop(0, _SKW // 16, vec, 0, unroll=8)
            pltpu.sync_copy(outv, out_hbm.at[b, c, pl.ds(base, _SKW)])
            return 0

        lax.fori_loop(0, _CH, chan, 0)

    return k


def kernel(xyz, features):
    B, N, _ = xyz.shape
    C = features.shape[1]
    S, K = N_FPS_, N_KNN_

    center_idx, fxyz_t = _fps_pallas(jax.lax.stop_gradient(xyz))
    FPS_xyz = fxyz_t.transpose(0, 2, 1)
    idx = _knn_pallas(fxyz_t, xyz)

    pf = jnp.concatenate([xyz.transpose(0, 2, 1), features], axis=1)
    idxf = idx.reshape(B, S * K)
    cidxf = jnp.repeat(center_idx, K, axis=1)
    out = _sc_gather_sub(B, N)(pf, idxf, cidxf)
    return (FPS_xyz, out.reshape(B, _CH, S, K))


# speedup vs baseline: 11.1390x; 1.2347x over previous
import functools

import jax
import jax.numpy as jnp
from jax.experimental import pallas as pl
from jax.experimental.pallas import tpu as pltpu

N_FPS_ = 2048
N_KNN_ = 20
_NR, _NL = 64, 128


def _fps_body_all(x_ref, xs_ref, idx_ref, fxyz_ref, dists_ref):
    B = idx_ref.shape[0]
    rows = jax.lax.broadcasted_iota(jnp.int32, (_NR, _NL), 0)
    cols = jax.lax.broadcasted_iota(jnp.int32, (_NR, _NL), 1)
    lin = rows * _NL + cols
    big = jnp.int32(2 ** 30)
    for b in range(B):
        dists_ref[b] = jnp.full((_NR, _NL), 1e10, jnp.float32)

    def body(i, fs):
        nfs = []
        for b in range(B):
            f = fs[b]
            x = x_ref[0, b]
            y = x_ref[1, b]
            z = x_ref[2, b]
            cx = xs_ref[0, b, f]
            cy = xs_ref[1, b, f]
            cz = xs_ref[2, b, f]
            idx_ref[b, 0, i] = f
            fxyz_ref[b, 0, i] = cx
            fxyz_ref[b, 1, i] = cy
            fxyz_ref[b, 2, i] = cz
            dx = x - cx
            dy = y - cy
            dz = z - cz
            d = (dx * dx + dy * dy) + dz * dz
            nd = jnp.minimum(dists_ref[b], d)
            dists_ref[b] = nd
            m = jnp.max(nd)
            nfs.append(jnp.min(jnp.where(nd == m, lin, big)))
        return tuple(nfs)

    jax.lax.fori_loop(0, N_FPS_, body, (jnp.int32(0),) * B)


def _fps_pallas(xyz):
    B, N, _ = xyz.shape
    xt = xyz.reshape(B, _NR, _NL, 3).transpose(3, 0, 1, 2)
    xs = xyz.transpose(2, 0, 1)
    idx, fxyz = pl.pallas_call(
        _fps_body_all,
        grid=(1,),
        in_specs=[
            pl.BlockSpec((3, B, _NR, _NL), lambda i: (0, 0, 0, 0)),
            pl.BlockSpec((3, B, N), lambda i: (0, 0, 0),
                         memory_space=pltpu.SMEM),
        ],
        out_specs=[
            pl.BlockSpec((B, 1, N_FPS_), lambda i: (0, 0, 0),
                         memory_space=pltpu.SMEM),
            pl.BlockSpec((B, 3, N_FPS_), lambda i: (0, 0, 0),
                         memory_space=pltpu.SMEM),
        ],
        out_shape=[
            jax.ShapeDtypeStruct((B, 1, N_FPS_), jnp.int32),
            jax.ShapeDtypeStruct((B, 3, N_FPS_), jnp.float32),
        ],
        scratch_shapes=[pltpu.VMEM((B, _NR, _NL), jnp.float32)],
    )(xt, xs)
    return idx.reshape(B, N_FPS_), fxyz


_QT = 128


def _knn_body(p_ref, q_ref, idx_ref, d_ref):
    p = p_ref[0]
    q = q_ref[0]
    px, py, pz = p[0:1, :], p[1:2, :], p[2:3, :]
    qx, qy, qz = q[:, 0:1], q[:, 1:2], q[:, 2:3]
    p2 = (px * px + py * py) + pz * pz
    q2 = (qx * qx + qy * qy) + qz * qz
    bf, f32 = jnp.bfloat16, jnp.float32
    pxb, pyb, pzb = (px.astype(bf).astype(f32), py.astype(bf).astype(f32),
                     pz.astype(bf).astype(f32))
    qxb, qyb, qzb = (qx.astype(bf).astype(f32), qy.astype(bf).astype(f32),
                     qz.astype(bf).astype(f32))
    qp = (qxb * pxb + qyb * pyb) + qzb * pzb
    d_ref[...] = (q2 + p2) - 2.0 * qp
    lin = jax.lax.broadcasted_iota(jnp.int32, (_QT, 8192), 1)
    big = jnp.int32(2 ** 30)
    inf = jnp.float32(3e38)
    for kk in range(N_KNN_):
        d = d_ref[...]
        m = jnp.min(d, axis=1, keepdims=True)
        ai = jnp.min(jnp.where(d == m, lin, big), axis=1, keepdims=True)
        idx_ref[0, :, kk:kk + 1] = ai
        d_ref[...] = jnp.where(lin == ai, inf, d)


def _knn_pallas(fxyz_t, xyz):
    B, N, _ = xyz.shape
    S = N_FPS_
    pt = jnp.concatenate(
        [xyz.transpose(0, 2, 1),
         jnp.zeros((B, 5, N), jnp.float32)], axis=1)
    qt = jnp.concatenate(
        [fxyz_t.transpose(0, 2, 1),
         jnp.zeros((B, S, 5), jnp.float32)], axis=2)
    idx = pl.pallas_call(
        _knn_body,
        grid=(B, S // _QT),
        in_specs=[
            pl.BlockSpec((1, 8, N), lambda b, s: (b, 0, 0)),
            pl.BlockSpec((1, _QT, 8), lambda b, s: (b, s, 0)),
        ],
        out_specs=pl.BlockSpec((1, _QT, N_KNN_), lambda b, s: (b, s, 0)),
        out_shape=jax.ShapeDtypeStruct((B, S, N_KNN_), jnp.int32),
        scratch_shapes=[pltpu.VMEM((_QT, N), jnp.float32)],
    )(pt, qt)
    return idx


_CH = 131
_TPB = 8
_SKW = N_FPS_ * N_KNN_ // _TPB


def _sc_gather_sub(B, N):
    from jax import lax
    from jax.experimental.pallas import tpu_sc as plsc

    mesh = plsc.VectorSubcoreMesh(core_axis_name="c", subcore_axis_name="s")

    @functools.partial(
        pl.kernel, mesh=mesh,
        compiler_params=pltpu.CompilerParams(needs_layout_passes=False),
        out_type=jax.ShapeDtypeStruct((B, _CH, _TPB * _SKW), jnp.float32),
        scratch_types=[
            pltpu.VMEM((N,), jnp.float32),
            pltpu.VMEM((_SKW,), jnp.int32),
            pltpu.VMEM((_SKW,), jnp.int32),
            pltpu.VMEM((_SKW,), jnp.float32),
        ],
    )
    def k(pf_hbm, idxf_hbm, cidx_hbm, out_hbm, colv, idxv, cidxv, outv):
        wid = lax.axis_index("s") * 2 + lax.axis_index("c")
        b = wid // _TPB
        base = (wid % _TPB) * _SKW
        pltpu.sync_copy(idxf_hbm.at[b, pl.ds(base, _SKW)], idxv)
        pltpu.sync_copy(cidx_hbm.at[b, pl.ds(base, _SKW)], cidxv)

        def chan(c, _):
            pltpu.sync_copy(pf_hbm.at[b, c], colv)

            def vec(j, _):
                iv = idxv[pl.ds(j * 16, 16)]
                cv = cidxv[pl.ds(j * 16, 16)]
                a = plsc.load_gather(colv, [iv])
                ctr = plsc.load_gather(colv, [cv])
                outv[pl.ds(j * 16, 16)] = a - ctr
                return 0

            lax.fori_loop(0, _SKW // 16, vec, 0, unroll=8)
            pltpu.sync_copy(outv, out_hbm.at[b, c, pl.ds(base, _SKW)])
            return 0

        lax.fori_loop(0, _CH, chan, 0)

    return k


def kernel(xyz, features):
    B, N, _ = xyz.shape
    C = features.shape[1]
    S, K = N_FPS_, N_KNN_

    center_idx, fxyz_t = _fps_pallas(jax.lax.stop_gradient(xyz))
    FPS_xyz = fxyz_t.transpose(0, 2, 1)
    idx = _knn_pallas(fxyz_t, xyz)

    pf = jnp.concatenate([xyz.transpose(0, 2, 1), features], axis=1)
    idxf = idx.reshape(B, S * K)
    cidxf = jnp.repeat(center_idx, K, axis=1)
    out = _sc_gather_sub(B, N)(pf, idxf, cidxf)
    return (FPS_xyz, out.reshape(B, _CH, S, K))


# speedup vs baseline: 14.9360x; 1.3409x over previous
import functools

import jax
import jax.numpy as jnp
from jax.experimental import pallas as pl
from jax.experimental.pallas import tpu as pltpu

N_FPS_ = 2048
N_KNN_ = 20
_NR, _NL = 64, 128


def _fps_body_all(x_ref, xs_ref, idx_ref, fxyz_ref, dists_ref):
    B = idx_ref.shape[0]
    rows = jax.lax.broadcasted_iota(jnp.int32, (_NR, _NL), 0)
    cols = jax.lax.broadcasted_iota(jnp.int32, (_NR, _NL), 1)
    lin = rows * _NL + cols
    big = jnp.int32(2 ** 30)
    for b in range(B):
        dists_ref[b] = jnp.full((_NR, _NL), 1e10, jnp.float32)

    def body(i, fs):
        cs, nds = [], []
        for b in range(B):
            f = fs[b]
            cx = xs_ref[0, b, f]
            cy = xs_ref[1, b, f]
            cz = xs_ref[2, b, f]
            idx_ref[b, 0, i] = f
            fxyz_ref[b, 0, i] = cx
            fxyz_ref[b, 1, i] = cy
            fxyz_ref[b, 2, i] = cz
            cs.append((cx, cy, cz))
        for b in range(B):
            cx, cy, cz = cs[b]
            dx = x_ref[0, b] - cx
            dy = x_ref[1, b] - cy
            dz = x_ref[2, b] - cz
            d = (dx * dx + dy * dy) + dz * dz
            nd = jnp.minimum(dists_ref[b], d)
            dists_ref[b] = nd
            nds.append(nd)
        ts = []
        for b in range(B):
            t = jnp.maximum(nds[b][0:32], nds[b][32:64])
            t = jnp.maximum(t[0:16], t[16:32])
            t = jnp.maximum(t[0:8], t[8:16])
            ts.append(t)
        ms = [jnp.max(ts[b], axis=(0, 1), keepdims=True) for b in range(B)]
        ws = []
        for b in range(B):
            w = jnp.where(nds[b] == ms[b], lin, big)
            w = jnp.minimum(w[0:32], w[32:64])
            w = jnp.minimum(w[0:16], w[16:32])
            w = jnp.minimum(w[0:8], w[8:16])
            ws.append(w)
        return tuple(jnp.min(ws[b]) for b in range(B))

    jax.lax.fori_loop(0, N_FPS_, body, (jnp.int32(0),) * B)


def _fps_pallas(xyz):
    B, N, _ = xyz.shape
    xt = xyz.reshape(B, _NR, _NL, 3).transpose(3, 0, 1, 2)
    xs = xyz.transpose(2, 0, 1)
    idx, fxyz = pl.pallas_call(
        _fps_body_all,
        grid=(1,),
        in_specs=[
            pl.BlockSpec((3, B, _NR, _NL), lambda i: (0, 0, 0, 0)),
            pl.BlockSpec((3, B, N), lambda i: (0, 0, 0),
                         memory_space=pltpu.SMEM),
        ],
        out_specs=[
            pl.BlockSpec((B, 1, N_FPS_), lambda i: (0, 0, 0),
                         memory_space=pltpu.SMEM),
            pl.BlockSpec((B, 3, N_FPS_), lambda i: (0, 0, 0),
                         memory_space=pltpu.SMEM),
        ],
        out_shape=[
            jax.ShapeDtypeStruct((B, 1, N_FPS_), jnp.int32),
            jax.ShapeDtypeStruct((B, 3, N_FPS_), jnp.float32),
        ],
        scratch_shapes=[pltpu.VMEM((B, _NR, _NL), jnp.float32)],
    )(xt, xs)
    return idx.reshape(B, N_FPS_), fxyz


_QT = 128


def _knn_body(p_ref, q_ref, idx_ref, d_ref):
    p = p_ref[0]
    q = q_ref[0]
    px, py, pz = p[0:1, :], p[1:2, :], p[2:3, :]
    qx, qy, qz = q[:, 0:1], q[:, 1:2], q[:, 2:3]
    p2 = (px * px + py * py) + pz * pz
    q2 = (qx * qx + qy * qy) + qz * qz
    bf, f32 = jnp.bfloat16, jnp.float32
    pxb, pyb, pzb = (px.astype(bf).astype(f32), py.astype(bf).astype(f32),
                     pz.astype(bf).astype(f32))
    qxb, qyb, qzb = (qx.astype(bf).astype(f32), qy.astype(bf).astype(f32),
                     qz.astype(bf).astype(f32))
    qp = (qxb * pxb + qyb * pyb) + qzb * pzb
    d_ref[...] = (q2 + p2) - 2.0 * qp
    lin = jax.lax.broadcasted_iota(jnp.int32, (_QT, 8192), 1)
    big = jnp.int32(2 ** 30)
    inf = jnp.float32(3e38)
    for kk in range(N_KNN_):
        d = d_ref[...]
        m = jnp.min(d, axis=1, keepdims=True)
        ai = jnp.min(jnp.where(d == m, lin, big), axis=1, keepdims=True)
        idx_ref[0, :, kk:kk + 1] = ai
        d_ref[...] = jnp.where(lin == ai, inf, d)


def _knn_pallas(fxyz_t, xyz):
    B, N, _ = xyz.shape
    S = N_FPS_
    pt = jnp.concatenate(
        [xyz.transpose(0, 2, 1),
         jnp.zeros((B, 5, N), jnp.float32)], axis=1)
    qt = jnp.concatenate(
        [fxyz_t.transpose(0, 2, 1),
         jnp.zeros((B, S, 5), jnp.float32)], axis=2)
    idx = pl.pallas_call(
        _knn_body,
        grid=(B, S // _QT),
        in_specs=[
            pl.BlockSpec((1, 8, N), lambda b, s: (b, 0, 0)),
            pl.BlockSpec((1, _QT, 8), lambda b, s: (b, s, 0)),
        ],
        out_specs=pl.BlockSpec((1, _QT, N_KNN_), lambda b, s: (b, s, 0)),
        out_shape=jax.ShapeDtypeStruct((B, S, N_KNN_), jnp.int32),
        scratch_shapes=[pltpu.VMEM((_QT, N), jnp.float32)],
    )(pt, qt)
    return idx


_CH = 131
_TPB = 8
_SKW = N_FPS_ * N_KNN_ // _TPB


def _sc_gather_sub(B, N):
    from jax import lax
    from jax.experimental.pallas import tpu_sc as plsc

    mesh = plsc.VectorSubcoreMesh(core_axis_name="c", subcore_axis_name="s")

    @functools.partial(
        pl.kernel, mesh=mesh,
        compiler_params=pltpu.CompilerParams(needs_layout_passes=False),
        out_type=jax.ShapeDtypeStruct((B, _CH, _TPB * _SKW), jnp.float32),
        scratch_types=[
            pltpu.VMEM((N,), jnp.float32),
            pltpu.VMEM((_SKW,), jnp.int32),
            pltpu.VMEM((_SKW,), jnp.int32),
            pltpu.VMEM((_SKW,), jnp.float32),
        ],
    )
    def k(pf_hbm, idxf_hbm, cidx_hbm, out_hbm, colv, idxv, cidxv, outv):
        wid = lax.axis_index("s") * 2 + lax.axis_index("c")
        b = wid // _TPB
        base = (wid % _TPB) * _SKW
        pltpu.sync_copy(idxf_hbm.at[b, pl.ds(base, _SKW)], idxv)
        pltpu.sync_copy(cidx_hbm.at[b, pl.ds(base, _SKW)], cidxv)

        def chan(c, _):
            pltpu.sync_copy(pf_hbm.at[b, c], colv)

            def vec(j, _):
                iv = idxv[pl.ds(j * 16, 16)]
                cv = cidxv[pl.ds(j * 16, 16)]
                a = plsc.load_gather(colv, [iv])
                ctr = plsc.load_gather(colv, [cv])
                outv[pl.ds(j * 16, 16)] = a - ctr
                return 0

            lax.fori_loop(0, _SKW // 16, vec, 0, unroll=8)
            pltpu.sync_copy(outv, out_hbm.at[b, c, pl.ds(base, _SKW)])
            return 0

        lax.fori_loop(0, _CH, chan, 0)

    return k


def kernel(xyz, features):
    B, N, _ = xyz.shape
    C = features.shape[1]
    S, K = N_FPS_, N_KNN_

    center_idx, fxyz_t = _fps_pallas(jax.lax.stop_gradient(xyz))
    FPS_xyz = fxyz_t.transpose(0, 2, 1)
    idx = _knn_pallas(fxyz_t, xyz)

    pf = jnp.concatenate([xyz.transpose(0, 2, 1), features], axis=1)
    idxf = idx.reshape(B, S * K)
    cidxf = jnp.repeat(center_idx, K, axis=1)
    out = _sc_gather_sub(B, N)(pf, idxf, cidxf)
    return (FPS_xyz, out.reshape(B, _CH, S, K))
